# baseline (speedup 1.0000x reference)
import functools

import jax
import jax.numpy as jnp
from jax import lax
from jax.experimental import pallas as pl
from jax.experimental.pallas import tpu as pltpu
from jax.experimental.pallas import tpu_sc as plsc

_NC = 2
_NS = 16
_NW = _NC * _NS
_BLK = 128
_CBLKS = 32


def _vgather(tab, idx):
    return lax.gather(
        tab, idx[:, None],
        lax.GatherDimensionNumbers(
            offset_dims=(), collapsed_slice_dims=(0,), start_index_map=(0,)),
        (1,), mode=lax.GatherScatterMode.PROMISE_IN_BOUNDS)


def _rsqrt_seed(s):
    i = lax.bitcast_convert_type(s, jnp.int32)
    return lax.bitcast_convert_type(
        jnp.int32(0x5F3759DF) - (i >> 1), jnp.float32)


def _rsqrt2(s):
    y = _rsqrt_seed(s)
    hs = 0.5 * s
    y = y * (1.5 - hs * y * y)
    y = y * (1.5 - hs * y * y)
    return y


def _rsqrt1(s):
    y = _rsqrt_seed(s)
    y = y * (1.5 - (0.5 * s) * y * y)
    return y


def _sc_body(blocks_per_worker, qh, ch, sh, out_h,
             inb, outb, ctab_v, stab_v, isem0, isem1, osem0, osem1):
    iters = blocks_per_worker // _CBLKS
    wid = lax.axis_index("c") * _NS + lax.axis_index("s")
    base_w = wid * blocks_per_worker
    isems = (isem0, isem1)
    osems = (osem0, osem1)

    pltpu.sync_copy(ch, ctab_v)
    pltpu.sync_copy(sh, stab_v)
    ctv = ctab_v[...]
    stv = stab_v[...]
    c8v = _vgather(ctv, jnp.full((16,), 8, jnp.int32))

    def in_cp(gg, b):
        off = base_w + gg * _CBLKS
        return pltpu.make_async_copy(
            qh.at[pl.ds(off, _CBLKS)], inb.at[b], isems[b])

    def out_cp(gg, b):
        off = base_w + gg * _CBLKS
        return pltpu.make_async_copy(
            outb.at[b], out_h.at[pl.ds(off, _CBLKS)], osems[b])

    in_cp(0, 0).start()

    def pair_body(h, carry):
        g = h * 2
        for b in range(2):
            gg = g + b

            @pl.when(gg + 1 < iters)
            def _():
                in_cp(gg + 1, 1 - b).start()

            in_cp(gg, b).wait()

            @pl.when(gg >= 2)
            def _():
                out_cp(gg - 2, b).wait()

            @plsc.parallel_loop(0, _CBLKS * (_BLK // 16), unroll=2)
            def blk(j):
                i = j >> 3
                k = j & 7
                if True:
                    d = pl.ds(k * 16, 16)
                    w = inb[b, i, 0, d]
                    x = inb[b, i, 1, d]
                    y = inb[b, i, 2, d]
                    z = inb[b, i, 3, d]

                    ww = w * w
                    sv = x * x + y * y + z * z
                    s = sv + ww
                    rs = _rsqrt2(s)
                    norms = jnp.maximum(s * rs, 1e-6)
                    w_u = w * jnp.minimum(rs, 1e6)

                    lo = jnp.where(w_u < c8v, 8, 0)
                    for sz in (4, 2, 1):
                        cm = _vgather(ctv, lo + sz)
                        lo = jnp.where(w_u < cm, lo + sz, lo)

                    cq = plsc.load_gather(ctab_v, [lo])
                    sq = plsc.load_gather(stab_v, [lo])
                    t = sq * (norms * _rsqrt2(sv))

                    outb[b, i, 0, d] = cq * norms
                    outb[b, i, 1, d] = x * t
                    outb[b, i, 2, d] = y * t
                    outb[b, i, 3, d] = z * t

            out_cp(gg, b).start()
        return carry

    lax.fori_loop(0, iters // 2, pair_body, 0)
    out_cp(iters - 2, 0).wait()
    out_cp(iters - 1, 1).wait()


@functools.partial(jax.jit, static_argnums=(3,))
def _run_sc(qv, ctab, stab, n_blocks):
    blocks_per_worker = n_blocks // _NW
    mesh = plsc.VectorSubcoreMesh(
        core_axis_name="c", subcore_axis_name="s",
        num_cores=_NC, num_subcores=_NS)
    f = pl.kernel(
        functools.partial(_sc_body, blocks_per_worker),
        out_type=jax.ShapeDtypeStruct((n_blocks, 4, _BLK), jnp.float32),
        mesh=mesh,
        scratch_types=[
            pltpu.VMEM((2, _CBLKS, 4, _BLK), jnp.float32),
            pltpu.VMEM((2, _CBLKS, 4, _BLK), jnp.float32),
            pltpu.VMEM((16,), jnp.float32),
            pltpu.VMEM((16,), jnp.float32),
            pltpu.SemaphoreType.DMA,
            pltpu.SemaphoreType.DMA,
            pltpu.SemaphoreType.DMA,
            pltpu.SemaphoreType.DMA,
        ],
        compiler_params=pltpu.CompilerParams(
            needs_layout_passes=False, use_tc_tiling_on_sc=False),
    )
    return f(qv, ctab, stab)


_TBT = 128


def _tc_body(ct_ref, st_ref, in_ref, out_ref):
    ct = ct_ref[0]
    st = st_ref[0]
    a = in_ref[...]
    qa = a.reshape(_TBT, 4, 128)
    w = qa[:, 0, :]
    x = qa[:, 1, :]
    y = qa[:, 2, :]
    z = qa[:, 3, :]

    sv = x * x + y * y + z * z
    s = sv + w * w
    rs = jnp.minimum(lax.rsqrt(s), 1e6)
    norms = jnp.maximum(s * rs, 1e-6)
    w_u = w * rs

    lo = jnp.zeros(w.shape, jnp.int32)
    for i in range(1, 16):
        lo = lo + (w_u < ct[i]).astype(jnp.int32)
    cq = jnp.full(w.shape, ct[0])
    sq = jnp.full(w.shape, st[0])
    for j in range(1, 16):
        m = lo == j
        cq = jnp.where(m, ct[j], cq)
        sq = jnp.where(m, st[j], sq)

    t = sq * (norms * jnp.minimum(lax.rsqrt(sv), 1e6))
    o = jnp.stack([cq * norms, x * t, y * t, z * t], axis=1)
    out_ref[...] = o.reshape(_TBT * 4, 128)


@functools.partial(jax.jit, static_argnums=(3,))
def _run_tc(qf, ct2, st2, n_blocks):
    grid = n_blocks // _TBT
    return pl.pallas_call(
        _tc_body,
        grid=(grid,),
        in_specs=[
            pl.BlockSpec((1, 16), lambda i: (0, 0)),
            pl.BlockSpec((1, 16), lambda i: (0, 0)),
            pl.BlockSpec((_TBT * 4, 128), lambda i: (i, 0)),
        ],
        out_specs=pl.BlockSpec((_TBT * 4, 128), lambda i: (i, 0)),
        out_shape=jax.ShapeDtypeStruct((n_blocks * 4, 128), jnp.float32),
    )(ct2, st2, qf)


def kernel(q_weights, bins):
    n_rows = q_weights.shape[0]
    n_blocks = n_rows // _BLK
    half = bins * 0.5
    ctab = jnp.cos(half)
    stab = jnp.sin(half)
    qv = q_weights.reshape(n_blocks, _BLK, 4).transpose(0, 2, 1)
    outv = _run_sc(qv, ctab, stab, n_blocks)
    return outv.transpose(0, 2, 1).reshape(n_rows, 4)


# speedup vs baseline: 2.0127x; 2.0127x over previous
import functools

import jax
import jax.numpy as jnp
from jax import lax
from jax.experimental import pallas as pl
from jax.experimental.pallas import tpu as pltpu
from jax.experimental.pallas import tpu_sc as plsc

_NC = 2
_NS = 16
_NW = _NC * _NS
_BLK = 128
_CBLKS = 32


def _vgather(tab, idx):
    return lax.gather(
        tab, idx[:, None],
        lax.GatherDimensionNumbers(
            offset_dims=(), collapsed_slice_dims=(0,), start_index_map=(0,)),
        (1,), mode=lax.GatherScatterMode.PROMISE_IN_BOUNDS)


def _rsqrt_seed(s):
    i = lax.bitcast_convert_type(s, jnp.int32)
    return lax.bitcast_convert_type(
        jnp.int32(0x5F3759DF) - (i >> 1), jnp.float32)


def _rsqrt2(s):
    y = _rsqrt_seed(s)
    hs = 0.5 * s
    y = y * (1.5 - hs * y * y)
    y = y * (1.5 - hs * y * y)
    return y


def _rsqrt1(s):
    y = _rsqrt_seed(s)
    y = y * (1.5 - (0.5 * s) * y * y)
    return y


def _sc_body(blocks_per_worker, qh, ch, sh, out_h,
             inb, outb, ctab_v, stab_v, isem0, isem1, osem0, osem1):
    iters = blocks_per_worker // _CBLKS
    wid = lax.axis_index("c") * _NS + lax.axis_index("s")
    base_w = wid * blocks_per_worker
    isems = (isem0, isem1)
    osems = (osem0, osem1)

    pltpu.sync_copy(ch, ctab_v)
    pltpu.sync_copy(sh, stab_v)
    ctv = ctab_v[...]
    stv = stab_v[...]
    c8v = _vgather(ctv, jnp.full((16,), 8, jnp.int32))

    def in_cp(gg, b):
        off = base_w + gg * _CBLKS
        return pltpu.make_async_copy(
            qh.at[pl.ds(off, _CBLKS)], inb.at[b], isems[b])

    def out_cp(gg, b):
        off = base_w + gg * _CBLKS
        return pltpu.make_async_copy(
            outb.at[b], out_h.at[pl.ds(off, _CBLKS)], osems[b])

    in_cp(0, 0).start()

    def pair_body(h, carry):
        g = h * 2
        for b in range(2):
            gg = g + b

            @pl.when(gg + 1 < iters)
            def _():
                in_cp(gg + 1, 1 - b).start()

            in_cp(gg, b).wait()

            @pl.when(gg >= 2)
            def _():
                out_cp(gg - 2, b).wait()

            @plsc.parallel_loop(0, _CBLKS * (_BLK // 16), unroll=2)
            def blk(j):
                i = j >> 3
                k = j & 7
                d = pl.ds(k * 16, 16)
                w = inb[b, i, 0, d]
                x = inb[b, i, 1, d]
                y = inb[b, i, 2, d]
                z = inb[b, i, 3, d]

                ww = w * w
                sv = x * x + y * y + z * z
                s = sv + ww
                rs = _rsqrt2(s)
                norms = jnp.maximum(s * rs, 1e-6)
                w_u = w * jnp.minimum(rs, 1e6)

                lo = jnp.where(w_u < c8v, 8, 0)
                for sz in (4, 2, 1):
                    cm = _vgather(ctv, lo + sz)
                    lo = jnp.where(w_u < cm, lo + sz, lo)

                cq = plsc.load_gather(ctab_v, [lo])
                sq = plsc.load_gather(stab_v, [lo])
                t = sq * (norms * _rsqrt1(sv))

                outb[b, i, 0, d] = cq * norms
                outb[b, i, 1, d] = x * t
                outb[b, i, 2, d] = y * t
                outb[b, i, 3, d] = z * t

            out_cp(gg, b).start()
        return carry

    lax.fori_loop(0, iters // 2, pair_body, 0)
    out_cp(iters - 2, 0).wait()
    out_cp(iters - 1, 1).wait()


@functools.partial(jax.jit, static_argnums=(3,))
def _run_sc(qv, ctab, stab, n_blocks):
    blocks_per_worker = n_blocks // _NW
    mesh = plsc.VectorSubcoreMesh(
        core_axis_name="c", subcore_axis_name="s",
        num_cores=_NC, num_subcores=_NS)
    f = pl.kernel(
        functools.partial(_sc_body, blocks_per_worker),
        out_type=jax.ShapeDtypeStruct((n_blocks, 4, _BLK), jnp.float32),
        mesh=mesh,
        scratch_types=[
            pltpu.VMEM((2, _CBLKS, 4, _BLK), jnp.float32),
            pltpu.VMEM((2, _CBLKS, 4, _BLK), jnp.float32),
            pltpu.VMEM((16,), jnp.float32),
            pltpu.VMEM((16,), jnp.float32),
            pltpu.SemaphoreType.DMA,
            pltpu.SemaphoreType.DMA,
            pltpu.SemaphoreType.DMA,
            pltpu.SemaphoreType.DMA,
        ],
        compiler_params=pltpu.CompilerParams(
            needs_layout_passes=False, use_tc_tiling_on_sc=False),
    )
    return f(qv, ctab, stab)


def kernel(q_weights, bins):
    n_rows = q_weights.shape[0]
    n_blocks = n_rows // _BLK
    half = bins * 0.5
    ctab = jnp.cos(half)
    stab = jnp.sin(half)
    qv = q_weights.reshape(n_blocks, _BLK, 4).transpose(0, 2, 1)
    outv = _run_sc(qv, ctab, stab, n_blocks)
    return outv.transpose(0, 2, 1).reshape(n_rows, 4)
